# baseline (speedup 1.0000x reference)
import math

import jax
import jax.numpy as jnp
from jax.experimental import pallas as pl
from jax.experimental.pallas import tpu as pltpu

Q_NUM = 50
MAX_CHOICES = 8
HEADS = 8
N_ANN = 100
EMB_DIM = 128
SENT_DIM = 384
FEAT_DIM = EMB_DIM + MAX_CHOICES + SENT_DIM
D_FF = 4 * FEAT_DIM
B, S = 16, 512
DH = FEAT_DIM // HEADS
HP = 128
HDIM = HEADS * HP
HALF = FEAT_DIM // 2
F32 = jnp.float32


def _dot(a, b, dims, precision=jax.lax.Precision.HIGHEST):
    return jax.lax.dot_general(a, b, (dims, ((), ())),
                               preferred_element_type=F32,
                               precision=precision)


def _ln(z, a, b):
    m = jnp.mean(z, axis=1, keepdims=True)
    d = z - m
    var = jnp.sum(d * d, axis=1, keepdims=True) / (FEAT_DIM - 1)
    return a * d / (jnp.sqrt(var) + 1e-6) + b


def _gather_body(qcol_ref, acol_ref, qtab_ref, atab_ref, out_ref):
    qc = qcol_ref[:, 0:1]
    ac = acol_ref[:, 0:1]
    qio = jax.lax.broadcasted_iota(jnp.int32, (1, Q_NUM), 1).astype(F32)
    aio = jax.lax.broadcasted_iota(jnp.int32, (1, N_ANN + 1), 1).astype(F32)
    qoh = (qc == qio).astype(F32)
    ai = jnp.where(ac < 0, float(N_ANN), ac)
    aoh = (ai == aio).astype(F32)
    out_ref[...] = (_dot(qoh, qtab_ref[...], ((1,), (0,))) +
                    _dot(aoh, atab_ref[...], ((1,), (0,))))


def _gather_call(qcol, acol, qtab, atab):
    return pl.pallas_call(
        _gather_body,
        grid=(B,),
        in_specs=[
            pl.BlockSpec((None, S, 8), lambda b: (b, 0, 0)),
            pl.BlockSpec((None, S, 8), lambda b: (b, 0, 0)),
            pl.BlockSpec((Q_NUM, EMB_DIM), lambda b: (0, 0)),
            pl.BlockSpec((N_ANN + 1, EMB_DIM), lambda b: (0, 0)),
        ],
        out_specs=pl.BlockSpec((None, S, EMB_DIM), lambda b: (b, 0, 0)),
        out_shape=jax.ShapeDtypeStruct((B, S, EMB_DIM), F32),
        compiler_params=pltpu.CompilerParams(
            dimension_semantics=("arbitrary",)),
    )(qcol, acol, qtab, atab)


def _layer_body(fx_ref, px_ref, qcol_ref, qrow_ref,
                wq_ref, qb_ref, wk_ref, kb_ref, wv_ref, vb_ref,
                ow_ref, ob_ref, n1a_ref, n1b_ref,
                f1w_ref, f1b_ref, f2w_ref, f2b_ref, n2a_ref, n2b_ref,
                puf_ref, pup_ref, pub_ref,
                s1w_ref, s1b_ref, s2w_ref, s2b_ref,
                c1w_ref, c1b_ref, c2w_ref, c2b_ref,
                fxo_ref, pxo_ref):
    fx = fx_ref[...]
    px = px_ref[...]

    q = _dot(fx, wq_ref[...], ((1,), (1,))) + qb_ref[0:1, :]
    k = _dot(fx, wk_ref[...], ((1,), (1,))) + kb_ref[0:1, :]
    v = _dot(fx, wv_ref[...], ((1,), (1,))) + vb_ref[0:1, :]
    scale = 1.0 / math.sqrt(DH)
    outs = []
    for h in range(HEADS):
        sl = slice(h * HP, (h + 1) * HP)
        sc = _dot(q[:, sl], k[:, sl], ((1,), (1,))) * scale
        sc = sc - jnp.max(sc, axis=1, keepdims=True)
        e = jnp.exp(sc)
        attn = e / jnp.sum(e, axis=1, keepdims=True)
        outs.append(_dot(attn, v[:, sl], ((1,), (0,))))
    out_all = jnp.concatenate(outs, axis=1)
    attn_out = _dot(out_all, ow_ref[...], ((1,), (1,))) + ob_ref[0:1, :]

    fx1 = _ln(fx + attn_out, n1a_ref[0:1, :], n1b_ref[0:1, :])
    ffh = jnp.maximum(_dot(fx1, f1w_ref[...], ((1,), (1,))) + f1b_ref[0:1, :],
                      0.0)
    ff = _dot(ffh, f2w_ref[...], ((1,), (1,))) + f2b_ref[0:1, :]
    fx2 = _ln(fx1 + ff, n2a_ref[0:1, :], n2b_ref[0:1, :])
    fxo_ref[...] = fx2

    pxl = (_dot(fx2, puf_ref[...], ((1,), (1,))) +
           _dot(px, pup_ref[...], ((1,), (1,))) + pub_ref[0:1, :])

    simh = jnp.maximum(_dot(fx2, s1w_ref[...], ((1,), (1,))) + s1b_ref[0:1, :],
                       0.0)
    sim = (jnp.sum(simh * s2w_ref[0:1, :], axis=1, keepdims=True) +
           s2b_ref[0:1, 0:1])
    confh = jnp.maximum(_dot(fx2, c1w_ref[...], ((1,), (1,))) +
                        c1b_ref[0:1, :], 0.0)
    conf = jax.nn.sigmoid(
        jnp.sum(confh * c2w_ref[0:1, :], axis=1, keepdims=True) +
        c2b_ref[0:1, 0:1])

    qc = qcol_ref[:, 0:1]
    qr = qrow_ref[0:1, :]
    mask = (qc == qr).astype(F32)
    sm = sim * mask
    sm = sm - jnp.max(sm, axis=0, keepdims=True)
    e = jnp.exp(sm)
    aw = e / jnp.sum(e, axis=0, keepdims=True)
    smoothed = _dot(aw, pxl, ((0,), (0,)))
    pxo_ref[...] = conf * pxl + (1.0 - conf) * smoothed


def _full(shape):
    return pl.BlockSpec(shape, lambda b: (0,) * len(shape))


def _vec8(v):
    v = jnp.asarray(v, F32).reshape(1, -1)
    return jnp.broadcast_to(v, (8, v.shape[1]))


def _prep_layer(p):
    pad = HP - DH
    wq = jnp.pad(p['Qw'].reshape(HEADS, DH, FEAT_DIM),
                 ((0, 0), (0, pad), (0, 0))).reshape(HDIM, FEAT_DIM)
    wk = jnp.pad(p['Kw'].reshape(HEADS, DH, FEAT_DIM),
                 ((0, 0), (0, pad), (0, 0))).reshape(HDIM, FEAT_DIM)
    wv = jnp.pad(p['Vw'].reshape(HEADS, DH, FEAT_DIM),
                 ((0, 0), (0, pad), (0, 0))).reshape(HDIM, FEAT_DIM)
    qb = _vec8(jnp.pad(p['Qb'].reshape(HEADS, DH),
                       ((0, 0), (0, pad))).reshape(HDIM))
    kb = _vec8(jnp.pad(p['Kb'].reshape(HEADS, DH),
                       ((0, 0), (0, pad))).reshape(HDIM))
    vb = _vec8(jnp.pad(p['Vb'].reshape(HEADS, DH),
                       ((0, 0), (0, pad))).reshape(HDIM))
    ow = jnp.pad(p['Ow'].reshape(FEAT_DIM, HEADS, DH),
                 ((0, 0), (0, 0), (0, pad))).reshape(FEAT_DIM, HDIM)
    return (wq, qb, wk, kb, wv, vb,
            ow, _vec8(p['Ob']), _vec8(p['n1a']), _vec8(p['n1b']),
            p['ff1w'], _vec8(p['ff1b']), p['ff2w'], _vec8(p['ff2b']),
            _vec8(p['n2a']), _vec8(p['n2b']),
            p['puw'][:, :FEAT_DIM], p['puw'][:, FEAT_DIM:], _vec8(p['pub']),
            p['sim1w'], _vec8(p['sim1b']), _vec8(p['sim2w'][0]),
            _vec8(jnp.broadcast_to(p['sim2b'], (8,))),
            p['conf1w'], _vec8(p['conf1b']), _vec8(p['conf2w'][0]),
            _vec8(jnp.broadcast_to(p['conf2b'], (8,))))


def _layer_call(fx, px, qcol, qrow, wts):
    in_specs = [
        pl.BlockSpec((None, S, FEAT_DIM), lambda b: (b, 0, 0)),
        pl.BlockSpec((None, S, MAX_CHOICES), lambda b: (b, 0, 0)),
        pl.BlockSpec((None, S, 8), lambda b: (b, 0, 0)),
        pl.BlockSpec((None, 8, S), lambda b: (b, 0, 0)),
    ] + [_full(w.shape) for w in wts]
    return pl.pallas_call(
        _layer_body,
        grid=(B,),
        in_specs=in_specs,
        out_specs=[
            pl.BlockSpec((None, S, FEAT_DIM), lambda b: (b, 0, 0)),
            pl.BlockSpec((None, S, MAX_CHOICES), lambda b: (b, 0, 0)),
        ],
        out_shape=[
            jax.ShapeDtypeStruct((B, S, FEAT_DIM), F32),
            jax.ShapeDtypeStruct((B, S, MAX_CHOICES), F32),
        ],
        compiler_params=pltpu.CompilerParams(
            dimension_semantics=("arbitrary",)),
    )(fx, px, qcol, qrow, *wts)


def kernel(x, annotators, questions, embeddings, annotator_embedding,
           question_embedding, layer_params):
    qf = questions.astype(F32)
    af = annotators.astype(F32)
    qcol = jnp.broadcast_to(qf[:, :, None], (B, S, 8))
    qrow = jnp.broadcast_to(qf[:, None, :], (B, 8, S))
    acol = jnp.broadcast_to(af[:, :, None], (B, S, 8))
    emb_sum = _gather_call(qcol, acol, question_embedding,
                           annotator_embedding)
    fx = jnp.concatenate([emb_sum, embeddings, x[:, :, 1:]], axis=-1)
    px = x[:, :, 1:]
    for p in layer_params:
        fx, px = _layer_call(fx, px, qcol, qrow, _prep_layer(p))
    return px


# speedup vs baseline: 4.8984x; 4.8984x over previous
import math

import jax
import jax.numpy as jnp
from jax.experimental import pallas as pl
from jax.experimental.pallas import tpu as pltpu

Q_NUM = 50
MAX_CHOICES = 8
HEADS = 8
N_ANN = 100
EMB_DIM = 128
SENT_DIM = 384
FEAT_DIM = EMB_DIM + MAX_CHOICES + SENT_DIM
D_FF = 4 * FEAT_DIM
B, S = 16, 512
DH = FEAT_DIM // HEADS
HP = 128
HDIM = HEADS * HP
HALF = FEAT_DIM // 2
F32 = jnp.float32


def _dot(a, b, dims, precision=jax.lax.Precision.DEFAULT):
    return jax.lax.dot_general(a.astype(jnp.bfloat16), b.astype(jnp.bfloat16),
                               (dims, ((), ())),
                               preferred_element_type=F32,
                               precision=precision)


def _ln(z, a, b):
    m = jnp.mean(z, axis=1, keepdims=True)
    d = z - m
    var = jnp.sum(d * d, axis=1, keepdims=True) / (FEAT_DIM - 1)
    return a * d / (jnp.sqrt(var) + 1e-6) + b


def _gather_body(qcol_ref, acol_ref, qtab_ref, atab_ref, out_ref):
    qc = qcol_ref[:, 0:1]
    ac = acol_ref[:, 0:1]
    qio = jax.lax.broadcasted_iota(jnp.int32, (1, Q_NUM), 1).astype(F32)
    aio = jax.lax.broadcasted_iota(jnp.int32, (1, N_ANN + 1), 1).astype(F32)
    qoh = (qc == qio).astype(F32)
    ai = jnp.where(ac < 0, float(N_ANN), ac)
    aoh = (ai == aio).astype(F32)
    out_ref[...] = (_dot(qoh, qtab_ref[...], ((1,), (0,))) +
                    _dot(aoh, atab_ref[...], ((1,), (0,))))


def _gather_call(qcol, acol, qtab, atab):
    return pl.pallas_call(
        _gather_body,
        grid=(B,),
        in_specs=[
            pl.BlockSpec((None, S, 8), lambda b: (b, 0, 0)),
            pl.BlockSpec((None, S, 8), lambda b: (b, 0, 0)),
            pl.BlockSpec((Q_NUM, EMB_DIM), lambda b: (0, 0)),
            pl.BlockSpec((N_ANN + 1, EMB_DIM), lambda b: (0, 0)),
        ],
        out_specs=pl.BlockSpec((None, S, EMB_DIM), lambda b: (b, 0, 0)),
        out_shape=jax.ShapeDtypeStruct((B, S, EMB_DIM), F32),
        compiler_params=pltpu.CompilerParams(
            dimension_semantics=("arbitrary",)),
    )(qcol, acol, qtab, atab)


def _layer_body(fx_ref, px_ref, qcol_ref, qrow_ref,
                wq_ref, qb_ref, wk_ref, kb_ref, wv_ref, vb_ref,
                ow_ref, ob_ref, n1a_ref, n1b_ref,
                f1w_ref, f1b_ref, f2w_ref, f2b_ref, n2a_ref, n2b_ref,
                puf_ref, pup_ref, pub_ref,
                s1w_ref, s1b_ref, s2w_ref, s2b_ref,
                c1w_ref, c1b_ref, c2w_ref, c2b_ref,
                fxo_ref, pxo_ref):
    fx = fx_ref[...]
    px = px_ref[...]

    q = _dot(fx, wq_ref[...], ((1,), (1,))) + qb_ref[0:1, :]
    k = _dot(fx, wk_ref[...], ((1,), (1,))) + kb_ref[0:1, :]
    v = _dot(fx, wv_ref[...], ((1,), (1,))) + vb_ref[0:1, :]
    scale = 1.0 / math.sqrt(DH)
    outs = []
    for h in range(HEADS):
        sl = slice(h * HP, (h + 1) * HP)
        sc = _dot(q[:, sl], k[:, sl], ((1,), (1,))) * scale
        sc = sc - jnp.max(sc, axis=1, keepdims=True)
        e = jnp.exp(sc)
        attn = e / jnp.sum(e, axis=1, keepdims=True)
        outs.append(_dot(attn, v[:, sl], ((1,), (0,))))
    out_all = jnp.concatenate(outs, axis=1)
    attn_out = _dot(out_all, ow_ref[...], ((1,), (1,))) + ob_ref[0:1, :]

    fx1 = _ln(fx + attn_out, n1a_ref[0:1, :], n1b_ref[0:1, :])
    ffh = jnp.maximum(_dot(fx1, f1w_ref[...], ((1,), (1,))) + f1b_ref[0:1, :],
                      0.0)
    ff = _dot(ffh, f2w_ref[...], ((1,), (1,))) + f2b_ref[0:1, :]
    fx2 = _ln(fx1 + ff, n2a_ref[0:1, :], n2b_ref[0:1, :])
    fxo_ref[...] = fx2

    pxl = (_dot(fx2, puf_ref[...], ((1,), (1,))) +
           _dot(px, pup_ref[...], ((1,), (1,))) + pub_ref[0:1, :])

    simh = jnp.maximum(_dot(fx2, s1w_ref[...], ((1,), (1,))) + s1b_ref[0:1, :],
                       0.0)
    sim = (jnp.sum(simh * s2w_ref[0:1, :], axis=1, keepdims=True) +
           s2b_ref[0:1, 0:1])
    confh = jnp.maximum(_dot(fx2, c1w_ref[...], ((1,), (1,))) +
                        c1b_ref[0:1, :], 0.0)
    conf = jax.nn.sigmoid(
        jnp.sum(confh * c2w_ref[0:1, :], axis=1, keepdims=True) +
        c2b_ref[0:1, 0:1])

    qc = qcol_ref[:, 0:1]
    qr = qrow_ref[0:1, :]
    mask = (qc == qr).astype(F32)
    sm = sim * mask
    sm = sm - jnp.max(sm, axis=0, keepdims=True)
    e = jnp.exp(sm)
    aw = e / jnp.sum(e, axis=0, keepdims=True)
    smoothed = _dot(aw, pxl, ((0,), (0,)))
    pxo_ref[...] = conf * pxl + (1.0 - conf) * smoothed


def _full(shape):
    return pl.BlockSpec(shape, lambda b: (0,) * len(shape))


def _vec8(v):
    v = jnp.asarray(v, F32).reshape(1, -1)
    return jnp.broadcast_to(v, (8, v.shape[1]))


def _prep_layer(p):
    pad = HP - DH
    wq = jnp.pad(p['Qw'].reshape(HEADS, DH, FEAT_DIM),
                 ((0, 0), (0, pad), (0, 0))).reshape(HDIM, FEAT_DIM)
    wk = jnp.pad(p['Kw'].reshape(HEADS, DH, FEAT_DIM),
                 ((0, 0), (0, pad), (0, 0))).reshape(HDIM, FEAT_DIM)
    wv = jnp.pad(p['Vw'].reshape(HEADS, DH, FEAT_DIM),
                 ((0, 0), (0, pad), (0, 0))).reshape(HDIM, FEAT_DIM)
    qb = _vec8(jnp.pad(p['Qb'].reshape(HEADS, DH),
                       ((0, 0), (0, pad))).reshape(HDIM))
    kb = _vec8(jnp.pad(p['Kb'].reshape(HEADS, DH),
                       ((0, 0), (0, pad))).reshape(HDIM))
    vb = _vec8(jnp.pad(p['Vb'].reshape(HEADS, DH),
                       ((0, 0), (0, pad))).reshape(HDIM))
    ow = jnp.pad(p['Ow'].reshape(FEAT_DIM, HEADS, DH),
                 ((0, 0), (0, 0), (0, pad))).reshape(FEAT_DIM, HDIM)
    return (wq, qb, wk, kb, wv, vb,
            ow, _vec8(p['Ob']), _vec8(p['n1a']), _vec8(p['n1b']),
            p['ff1w'], _vec8(p['ff1b']), p['ff2w'], _vec8(p['ff2b']),
            _vec8(p['n2a']), _vec8(p['n2b']),
            p['puw'][:, :FEAT_DIM], p['puw'][:, FEAT_DIM:], _vec8(p['pub']),
            p['sim1w'], _vec8(p['sim1b']), _vec8(p['sim2w'][0]),
            _vec8(jnp.broadcast_to(p['sim2b'], (8,))),
            p['conf1w'], _vec8(p['conf1b']), _vec8(p['conf2w'][0]),
            _vec8(jnp.broadcast_to(p['conf2b'], (8,))))


def _layer_call(fx, px, qcol, qrow, wts):
    in_specs = [
        pl.BlockSpec((None, S, FEAT_DIM), lambda b: (b, 0, 0)),
        pl.BlockSpec((None, S, MAX_CHOICES), lambda b: (b, 0, 0)),
        pl.BlockSpec((None, S, 8), lambda b: (b, 0, 0)),
        pl.BlockSpec((None, 8, S), lambda b: (b, 0, 0)),
    ] + [_full(w.shape) for w in wts]
    return pl.pallas_call(
        _layer_body,
        grid=(B,),
        in_specs=in_specs,
        out_specs=[
            pl.BlockSpec((None, S, FEAT_DIM), lambda b: (b, 0, 0)),
            pl.BlockSpec((None, S, MAX_CHOICES), lambda b: (b, 0, 0)),
        ],
        out_shape=[
            jax.ShapeDtypeStruct((B, S, FEAT_DIM), F32),
            jax.ShapeDtypeStruct((B, S, MAX_CHOICES), F32),
        ],
        compiler_params=pltpu.CompilerParams(
            dimension_semantics=("arbitrary",)),
    )(fx, px, qcol, qrow, *wts)


def kernel(x, annotators, questions, embeddings, annotator_embedding,
           question_embedding, layer_params):
    qf = questions.astype(F32)
    af = annotators.astype(F32)
    qcol = jnp.broadcast_to(qf[:, :, None], (B, S, 8))
    qrow = jnp.broadcast_to(qf[:, None, :], (B, 8, S))
    acol = jnp.broadcast_to(af[:, :, None], (B, S, 8))
    emb_sum = _gather_call(qcol, acol, question_embedding,
                           annotator_embedding)
    fx = jnp.concatenate([emb_sum, embeddings, x[:, :, 1:]], axis=-1)
    px = x[:, :, 1:]
    for p in layer_params:
        fx, px = _layer_call(fx, px, qcol, qrow, _prep_layer(p))
    return px
